# Initial kernel scaffold; baseline (speedup 1.0000x reference)
import jax
import jax.numpy as jnp
from jax.experimental import pallas as pl


def kernel(x, edge_index, edge_attr, W_edge, b_edge, W_res, b_res, W1, b1, W2, b2):
    raise NotImplementedError("write your pallas kernel here")


# speedup vs baseline: 2.1508x; 2.1508x over previous
import functools

import jax
import jax.numpy as jnp
from jax import lax
from jax.experimental import pallas as pl
from jax.experimental.pallas import tpu as pltpu
from jax.experimental.pallas import tpu_sc as plsc

N_NODES = 10000
N_EDGES = 320000
D_FEAT = 128
D_EDGE = 16

NC = 2
NS = 16
NW = NC * NS
EPW = N_EDGES // NW
CHUNK = 80
NCHUNK = EPW // CHUNK
PAD_NODES = 10240
ROWS_PER_TILE = PAD_NODES // NS


def _xr_body(x_ref, w_ref, o_ref):
    o_ref[...] = jnp.dot(x_ref[...], w_ref[...], preferred_element_type=jnp.float32)


def _compute_xr(x, w_top):
    bn = 2000
    return pl.pallas_call(
        _xr_body,
        grid=(N_NODES // bn,),
        in_specs=[
            pl.BlockSpec((bn, D_FEAT), lambda i: (i, 0)),
            pl.BlockSpec((D_FEAT, D_FEAT), lambda i: (0, 0)),
        ],
        out_specs=pl.BlockSpec((bn, D_FEAT), lambda i: (i, 0)),
        out_shape=jax.ShapeDtypeStruct((N_NODES, D_FEAT), jnp.float32),
    )(x, w_top)


def _eterm_body(ea_ref, we_ref, be_ref, wb_ref, br_ref, o_ref):
    h = jnp.dot(ea_ref[...], we_ref[...], preferred_element_type=jnp.float32)
    h = jnp.maximum(h + be_ref[...], 0.0)
    o_ref[...] = jnp.dot(h, wb_ref[...], preferred_element_type=jnp.float32) + br_ref[...]


def _compute_eterm(edge_attr, w_edge, b_edge, w_bot, b_res):
    be = 8000
    return pl.pallas_call(
        _eterm_body,
        grid=(N_EDGES // be,),
        in_specs=[
            pl.BlockSpec((be, D_EDGE), lambda i: (i, 0)),
            pl.BlockSpec((D_EDGE, D_EDGE), lambda i: (0, 0)),
            pl.BlockSpec((1, D_EDGE), lambda i: (0, 0)),
            pl.BlockSpec((D_EDGE, D_FEAT), lambda i: (0, 0)),
            pl.BlockSpec((1, D_FEAT), lambda i: (0, 0)),
        ],
        out_specs=pl.BlockSpec((be, D_FEAT), lambda i: (i, 0)),
        out_shape=jax.ShapeDtypeStruct((N_EDGES, D_FEAT), jnp.float32),
    )(edge_attr, w_edge, b_edge.reshape(1, D_EDGE), w_bot, b_res.reshape(1, D_FEAT))


def _sc_body(src_hbm, dst_hbm, xr_hbm, ete_hbm, zero_hbm, out_hbm,
             src_v, didx_v, rows_v, ete_v, agg_sh, sem):
    c = lax.axis_index("c")
    s = lax.axis_index("s")
    wid = c * NS + s

    pltpu.sync_copy(zero_hbm.at[pl.ds(s * ROWS_PER_TILE, ROWS_PER_TILE)],
                    agg_sh.at[pl.ds(s * ROWS_PER_TILE, ROWS_PER_TILE)])

    pltpu.sync_copy(src_hbm.at[wid], src_v)
    plsc.subcore_barrier()

    def chunk(j, carry):
        pltpu.sync_copy(dst_hbm.at[wid, j], didx_v)
        gcopy = pltpu.async_copy(xr_hbm.at[src_v.at[j]], rows_v, sem)
        pltpu.sync_copy(ete_hbm.at[wid, j], ete_v)
        gcopy.wait()

        def row(i, carry2):
            for k in range(D_FEAT // 16):
                v = rows_v[i, pl.ds(k * 16, 16)] + ete_v[i, pl.ds(k * 16, 16)]
                rows_v[i, pl.ds(k * 16, 16)] = jnp.maximum(v, 0.0)
            return carry2

        lax.fori_loop(0, CHUNK, row, 0, unroll=2)

        pltpu.sync_copy(rows_v, agg_sh.at[didx_v], add=True)
        return carry

    lax.fori_loop(0, NCHUNK, chunk, 0)

    plsc.subcore_barrier()

    pltpu.sync_copy(agg_sh.at[pl.ds(s * ROWS_PER_TILE, ROWS_PER_TILE)],
                    out_hbm.at[c, pl.ds(s * ROWS_PER_TILE, ROWS_PER_TILE)])


def _sc_aggregate(src_r, dst_r, xr, eterm_r, zeros):
    mesh = plsc.VectorSubcoreMesh(core_axis_name="c", subcore_axis_name="s",
                                  num_cores=NC, num_subcores=NS)
    f = pl.kernel(
        _sc_body,
        out_type=jax.ShapeDtypeStruct((NC, PAD_NODES, D_FEAT), jnp.float32),
        mesh=mesh,
        scratch_types=[
            pltpu.VMEM((NCHUNK, CHUNK), jnp.int32),
            pltpu.VMEM((CHUNK,), jnp.int32),
            pltpu.VMEM((CHUNK, D_FEAT), jnp.float32),
            pltpu.VMEM((CHUNK, D_FEAT), jnp.float32),
            pltpu.VMEM_SHARED((PAD_NODES, D_FEAT), jnp.float32),
            pltpu.SemaphoreType.DMA,
        ],
    )
    return f(src_r, dst_r, xr, eterm_r, zeros)


def _out_body(p0_ref, p1_ref, x_ref, w1_ref, b1_ref, w2_ref, b2_ref, o_ref):
    t = p0_ref[...] + p1_ref[...] + x_ref[...]
    h = jnp.dot(t, w1_ref[...], preferred_element_type=jnp.float32)
    h = jnp.maximum(h + b1_ref[...], 0.0)
    o_ref[...] = jnp.dot(h, w2_ref[...], preferred_element_type=jnp.float32) + b2_ref[...]


def _compute_out(p0, p1, x, w1, b1, w2, b2):
    bn = 2000
    return pl.pallas_call(
        _out_body,
        grid=(N_NODES // bn,),
        in_specs=[
            pl.BlockSpec((bn, D_FEAT), lambda i: (i, 0)),
            pl.BlockSpec((bn, D_FEAT), lambda i: (i, 0)),
            pl.BlockSpec((bn, D_FEAT), lambda i: (i, 0)),
            pl.BlockSpec((D_FEAT, D_FEAT), lambda i: (0, 0)),
            pl.BlockSpec((1, D_FEAT), lambda i: (0, 0)),
            pl.BlockSpec((D_FEAT, D_FEAT), lambda i: (0, 0)),
            pl.BlockSpec((1, D_FEAT), lambda i: (0, 0)),
        ],
        out_specs=pl.BlockSpec((bn, D_FEAT), lambda i: (i, 0)),
        out_shape=jax.ShapeDtypeStruct((N_NODES, D_FEAT), jnp.float32),
    )(p0, p1, x, w1, b1.reshape(1, D_FEAT), w2, b2.reshape(1, D_FEAT))


def kernel(x, edge_index, edge_attr, W_edge, b_edge, W_res, b_res, W1, b1, W2, b2):
    src = edge_index[0].astype(jnp.int32)
    dst = edge_index[1].astype(jnp.int32)
    w_top = W_res[:D_FEAT]
    w_bot = W_res[D_FEAT:]

    xr = _compute_xr(x, w_top)
    eterm = _compute_eterm(edge_attr, W_edge, b_edge, w_bot, b_res)

    src_r = src.reshape(NW, NCHUNK, CHUNK)
    dst_r = dst.reshape(NW, NCHUNK, CHUNK)
    eterm_r = eterm.reshape(NW, NCHUNK, CHUNK, D_FEAT)
    zeros = jnp.zeros((PAD_NODES, D_FEAT), jnp.float32)

    partial = _sc_aggregate(src_r, dst_r, xr, eterm_r, zeros)

    return _compute_out(partial[0, :N_NODES], partial[1, :N_NODES], x, W1, b1, W2, b2)
